# Initial kernel scaffold; baseline (speedup 1.0000x reference)
import jax
import jax.numpy as jnp
from jax.experimental import pallas as pl


def kernel(bonds, bond_atom_1, bond_atom_2, atoms, We1, be1, We2, be2, We3, be3, Wv1, bv1, Wv2, bv2, Wv3, bv3):
    raise NotImplementedError("write your pallas kernel here")


# speedup vs baseline: 3.3186x; 3.3186x over previous
import functools

import jax
import jax.numpy as jnp
from jax import lax
from jax.experimental import pallas as pl
from jax.experimental.pallas import tpu as pltpu
from jax.experimental.pallas import tpu_sc as plsc

NC = 2
NS = 16
NW = NC * NS
L = 16


@functools.lru_cache(maxsize=None)
def _make_gather(N, E, D):
  BLK = 512
  NBLK = E // BLK
  KMAX = -(-NBLK // NW)
  mesh = plsc.VectorSubcoreMesh(core_axis_name="c", subcore_axis_name="s")

  @functools.partial(
      pl.kernel,
      out_type=(jax.ShapeDtypeStruct((E, D), jnp.float32),
                jax.ShapeDtypeStruct((E, D), jnp.float32)),
      mesh=mesh,
      scratch_types=[
          pltpu.VMEM((4, 128), jnp.int32),
          pltpu.VMEM((4, 128), jnp.int32),
          pltpu.VMEM((BLK, D), jnp.float32),
          pltpu.VMEM((BLK, D), jnp.float32),
          pltpu.SemaphoreType.DMA,
      ],
      compiler_params=pltpu.CompilerParams(use_tc_tiling_on_sc=False),
  )
  def gk(atoms_hbm, idx1_hbm, idx2_hbm, out1, out2, i1v, i2v, r1v, r2v, sem):
    wid = lax.axis_index("s") * NC + lax.axis_index("c")

    def body(k, _):
      b = wid + k * NW

      @pl.when(b < NBLK)
      def _():
        pltpu.sync_copy(idx1_hbm.at[pl.ds(b * 4, 4)], i1v)
        pltpu.sync_copy(idx2_hbm.at[pl.ds(b * 4, 4)], i2v)
        cps = []
        for j in range(4):
          cps.append(pltpu.async_copy(
              atoms_hbm.at[i1v.at[j]], r1v.at[pl.ds(j * 128, 128)], sem))
          cps.append(pltpu.async_copy(
              atoms_hbm.at[i2v.at[j]], r2v.at[pl.ds(j * 128, 128)], sem))
        for cp in cps:
          cp.wait()
        pltpu.sync_copy(r1v, out1.at[pl.ds(b * BLK, BLK)])
        pltpu.sync_copy(r2v, out2.at[pl.ds(b * BLK, BLK)])
      return 0

    lax.fori_loop(0, KMAX, body, 0)

  return gk


def _edge_mlp_body(a1r, a2r, br, w1a, w1b, w1c, b1, w2, b2, w3, b3, outr):
  x = (jnp.dot(a1r[...], w1a[...], preferred_element_type=jnp.float32)
       + jnp.dot(a2r[...], w1b[...], preferred_element_type=jnp.float32)
       + jnp.dot(br[...], w1c[...], preferred_element_type=jnp.float32)
       + b1[...])
  h = jnp.maximum(x, 0.0)
  h = jnp.maximum(
      jnp.dot(h, w2[...], preferred_element_type=jnp.float32) + b2[...], 0.0)
  outr[...] = jnp.dot(h, w3[...], preferred_element_type=jnp.float32) + b3[...]


@functools.lru_cache(maxsize=None)
def _make_edge_mlp(E, D, B):
  G = E // B
  full = lambda s: pl.BlockSpec(s, lambda i: (0, 0))
  return pl.pallas_call(
      _edge_mlp_body,
      grid=(G,),
      in_specs=[
          pl.BlockSpec((B, D), lambda i: (i, 0)),
          pl.BlockSpec((B, D), lambda i: (i, 0)),
          pl.BlockSpec((B, D), lambda i: (i, 0)),
          full((D, 128)), full((D, 128)), full((D, 128)), full((1, 128)),
          full((128, 64)), full((1, 64)),
          full((64, D)), full((1, D)),
      ],
      out_specs=pl.BlockSpec((B, D), lambda i: (i, 0)),
      out_shape=jax.ShapeDtypeStruct((E, D), jnp.float32),
  )


@functools.lru_cache(maxsize=None)
def _make_scatter(N, E, D):
  CH = 512
  NCH = E // CH
  KMAX = -(-NCH // NS)
  NH = N // 2
  NHP = NH + 176
  STRIPE = NHP // NS
  ZR = 224
  mesh = plsc.VectorSubcoreMesh(core_axis_name="c", subcore_axis_name="s")

  @functools.partial(
      pl.kernel,
      out_type=(jax.ShapeDtypeStruct((NC, NHP, D), jnp.float32),
                jax.ShapeDtypeStruct((NC, NHP), jnp.float32)),
      mesh=mesh,
      scratch_types=[
          pltpu.VMEM((4, 128), jnp.int32),
          pltpu.VMEM((4, 128), jnp.int32),
          pltpu.VMEM((CH, D), jnp.float32),
          pltpu.VMEM((CH,), jnp.float32),
          pltpu.VMEM((ZR, D), jnp.float32),
          pltpu.VMEM((ZR,), jnp.float32),
          pltpu.VMEM_SHARED((NHP, D), jnp.float32),
          pltpu.VMEM_SHARED((NHP,), jnp.float32),
      ],
      compiler_params=pltpu.CompilerParams(use_tc_tiling_on_sc=False),
  )
  def sk(edges_hbm, idx_hbm, sums_out, cnt_out,
         iv, lv, pv, ov, zv, zcv, acc, accc):
    c = lax.axis_index("c")
    s = lax.axis_index("s")
    base = c * NH

    def fill_z(r, _):
      for g in range(D // L):
        zv[r, pl.ds(g * L, L)] = jnp.zeros((L,), jnp.float32)
      return 0
    lax.fori_loop(0, ZR, fill_z, 0)

    def fill_zc(r, _):
      zcv[pl.ds(r * L, L)] = jnp.zeros((L,), jnp.float32)
      return 0
    lax.fori_loop(0, ZR // L, fill_zc, 0)

    def fill_o(r, _):
      ov[pl.ds(r * L, L)] = jnp.ones((L,), jnp.float32)
      return 0
    lax.fori_loop(0, CH // L, fill_o, 0)

    for q in range(STRIPE // ZR):
      pltpu.sync_copy(zv, acc.at[pl.ds(s * STRIPE + q * ZR, ZR)])
      pltpu.sync_copy(zcv, accc.at[pl.ds(s * STRIPE + q * ZR, ZR)])
    plsc.subcore_barrier()

    def body(k, _):
      i = s + k * NS

      @pl.when(i < NCH)
      def _():
        pltpu.sync_copy(idx_hbm.at[pl.ds(i * 4, 4)], iv)
        pltpu.sync_copy(edges_hbm.at[pl.ds(i * CH, CH)], pv)
        for j in range(4):
          for g in range(128 // L):
            v = iv[j, pl.ds(g * L, L)] - base
            ok = (v >= 0) & (v < NH)
            lv[j, pl.ds(g * L, L)] = jnp.where(ok, v, NH)
        for j in range(4):
          pltpu.sync_copy(pv.at[pl.ds(j * 128, 128)], acc.at[lv.at[j]],
                          add=True)
          pltpu.sync_copy(ov.at[pl.ds(j * 128, 128)], accc.at[lv.at[j]],
                          add=True)
      return 0

    lax.fori_loop(0, KMAX, body, 0)
    plsc.subcore_barrier()

    pltpu.sync_copy(acc.at[pl.ds(s * STRIPE, STRIPE)],
                    sums_out.at[c].at[pl.ds(s * STRIPE, STRIPE)])
    pltpu.sync_copy(accc.at[pl.ds(s * STRIPE, STRIPE)],
                    cnt_out.at[c].at[pl.ds(s * STRIPE, STRIPE)])

  return sk


def _node_mlp_body(sr, cr, ar, w1a, w1b, b1, w2, b2, w3, b3, outr):
  cnt = jnp.maximum(cr[0], 1.0)
  mean = sr[0] / cnt
  x = (jnp.dot(mean, w1a[...], preferred_element_type=jnp.float32)
       + jnp.dot(ar[...], w1b[...], preferred_element_type=jnp.float32)
       + b1[...])
  h = jnp.maximum(x, 0.0)
  h = jnp.maximum(
      jnp.dot(h, w2[...], preferred_element_type=jnp.float32) + b2[...], 0.0)
  outr[...] = jnp.dot(h, w3[...], preferred_element_type=jnp.float32) + b3[...]


@functools.lru_cache(maxsize=None)
def _make_node_mlp(N, NHP, D, Bn):
  G = N // Bn
  PB = G // NC
  full = lambda s: pl.BlockSpec(s, lambda i: (0, 0))
  return pl.pallas_call(
      _node_mlp_body,
      grid=(G,),
      in_specs=[
          pl.BlockSpec((1, Bn, D), lambda i: (i // PB, i % PB, 0)),
          pl.BlockSpec((1, Bn, 1), lambda i: (i // PB, i % PB, 0)),
          pl.BlockSpec((Bn, D), lambda i: (i, 0)),
          full((D, 128)), full((D, 128)), full((1, 128)),
          full((128, 64)), full((1, 64)),
          full((64, D)), full((1, D)),
      ],
      out_specs=pl.BlockSpec((Bn, D), lambda i: (i, 0)),
      out_shape=jax.ShapeDtypeStruct((N, D), jnp.float32),
  )


def kernel(bonds, bond_atom_1, bond_atom_2, atoms,
           We1, be1, We2, be2, We3, be3,
           Wv1, bv1, Wv2, bv2, Wv3, bv3):
  E, D = bonds.shape
  N = atoms.shape[0]

  idx1 = bond_atom_1.astype(jnp.int32).reshape(E // 128, 128)
  idx2 = bond_atom_2.astype(jnp.int32).reshape(E // 128, 128)

  a1, a2 = _make_gather(N, E, D)(atoms, idx1, idx2)

  bonds_new = _make_edge_mlp(E, D, 4000)(
      a1, a2, bonds,
      We1[:D], We1[D:2 * D], We1[2 * D:], be1.reshape(1, -1),
      We2, be2.reshape(1, -1), We3, be3.reshape(1, -1))

  sums, cnt = _make_scatter(N, E, D)(bonds_new, idx2)
  NHP = sums.shape[1]

  atoms_new = _make_node_mlp(N, NHP, D, 1000)(
      sums, cnt.reshape(NC, NHP, 1), atoms,
      Wv1[:D], Wv1[D:], bv1.reshape(1, -1),
      Wv2, bv2.reshape(1, -1), Wv3, bv3.reshape(1, -1))

  return (atoms_new, bonds_new)


# speedup vs baseline: 5.8505x; 1.7629x over previous
import functools

import jax
import jax.numpy as jnp
from jax import lax
from jax.experimental import pallas as pl
from jax.experimental.pallas import tpu as pltpu
from jax.experimental.pallas import tpu_sc as plsc

NC = 2
NS = 16
NW = NC * NS
L = 16


@functools.lru_cache(maxsize=None)
def _make_gather(N, E, D):
  BLK = 512
  NBLK = E // BLK
  KMAX = -(-NBLK // NW)
  mesh = plsc.VectorSubcoreMesh(core_axis_name="c", subcore_axis_name="s")

  @functools.partial(
      pl.kernel,
      out_type=(jax.ShapeDtypeStruct((NBLK, BLK, D), jnp.float32),
                jax.ShapeDtypeStruct((NBLK, BLK, D), jnp.float32)),
      mesh=mesh,
      scratch_types=[
          pltpu.VMEM((4, 128), jnp.int32),
          pltpu.VMEM((4, 128), jnp.int32),
          pltpu.VMEM((BLK, D), jnp.float32),
          pltpu.VMEM((BLK, D), jnp.float32),
          pltpu.SemaphoreType.DMA,
      ],
      compiler_params=pltpu.CompilerParams(use_tc_tiling_on_sc=False),
  )
  def gk(atoms_hbm, idx1_hbm, idx2_hbm, out1, out2, i1v, i2v, r1v, r2v, sem):
    wid = lax.axis_index("s") * NC + lax.axis_index("c")

    def body(k, _):
      b = wid + k * NW

      @pl.when(b < NBLK)
      def _():
        pltpu.sync_copy(idx1_hbm.at[pl.ds(b * 4, 4)], i1v)
        pltpu.sync_copy(idx2_hbm.at[pl.ds(b * 4, 4)], i2v)
        cps = []
        for j in range(4):
          cps.append(pltpu.async_copy(
              atoms_hbm.at[i1v.at[j]], r1v.at[pl.ds(j * 128, 128)], sem))
          cps.append(pltpu.async_copy(
              atoms_hbm.at[i2v.at[j]], r2v.at[pl.ds(j * 128, 128)], sem))
        for cp in cps:
          cp.wait()
        pltpu.sync_copy(r1v, out1.at[b])
        pltpu.sync_copy(r2v, out2.at[b])
      return 0

    lax.fori_loop(0, KMAX, body, 0)

  return gk


def _edge_mlp_body(a1r, a2r, br, w1, b1, w2, b2, w3, b3, outr):
  x = jnp.concatenate([a1r[...], a2r[...], br[...]], axis=1)
  h = jnp.maximum(
      jnp.dot(x, w1[...], preferred_element_type=jnp.float32) + b1[...], 0.0)
  h = jnp.maximum(
      jnp.dot(h, w2[...], preferred_element_type=jnp.float32) + b2[...], 0.0)
  outr[...] = jnp.dot(h, w3[...], preferred_element_type=jnp.float32) + b3[...]


@functools.lru_cache(maxsize=None)
def _make_edge_mlp(EP, B):
  G = EP // B
  full = lambda s: pl.BlockSpec(s, lambda i: (0, 0))
  return pl.pallas_call(
      _edge_mlp_body,
      grid=(G,),
      in_specs=[
          pl.BlockSpec((B, 128), lambda i: (i, 0)),
          pl.BlockSpec((B, 128), lambda i: (i, 0)),
          pl.BlockSpec((B, 128), lambda i: (i, 0)),
          full((384, 512)), full((1, 512)),
          full((512, 256)), full((1, 256)),
          full((256, 128)), full((1, 128)),
      ],
      out_specs=pl.BlockSpec((B, 128), lambda i: (i, 0)),
      out_shape=jax.ShapeDtypeStruct((EP, 128), jnp.float32),
  )


def _blockdiag4(w):
  din, dout = w.shape
  out = jnp.zeros((4 * din, 4 * dout), w.dtype)
  for q in range(4):
    out = out.at[q * din:(q + 1) * din, q * dout:(q + 1) * dout].set(w)
  return out


@functools.lru_cache(maxsize=None)
def _make_scatter(N, E, D):
  CH = 512
  NCH = E // CH
  KMAX = -(-NCH // NS)
  NH = N // 2
  NHP = NH + 176
  STRIPE = NHP // NS
  ZR = 224
  mesh = plsc.VectorSubcoreMesh(core_axis_name="c", subcore_axis_name="s")

  @functools.partial(
      pl.kernel,
      out_type=(jax.ShapeDtypeStruct((NC, NHP, D), jnp.float32),
                jax.ShapeDtypeStruct((NC, NHP), jnp.float32)),
      mesh=mesh,
      scratch_types=[
          pltpu.VMEM((4, 128), jnp.int32),
          pltpu.VMEM((4, 128), jnp.int32),
          pltpu.VMEM((CH, D), jnp.float32),
          pltpu.VMEM((CH,), jnp.float32),
          pltpu.VMEM((ZR, D), jnp.float32),
          pltpu.VMEM((ZR,), jnp.float32),
          pltpu.VMEM_SHARED((NHP, D), jnp.float32),
          pltpu.VMEM_SHARED((NHP,), jnp.float32),
      ],
      compiler_params=pltpu.CompilerParams(use_tc_tiling_on_sc=False),
  )
  def sk(edges_hbm, idx_hbm, sums_out, cnt_out,
         iv, lv, pv, ov, zv, zcv, acc, accc):
    c = lax.axis_index("c")
    s = lax.axis_index("s")
    base = c * NH

    def fill_z(r, _):
      for g in range(D // L):
        zv[r, pl.ds(g * L, L)] = jnp.zeros((L,), jnp.float32)
      return 0
    lax.fori_loop(0, ZR, fill_z, 0)

    def fill_zc(r, _):
      zcv[pl.ds(r * L, L)] = jnp.zeros((L,), jnp.float32)
      return 0
    lax.fori_loop(0, ZR // L, fill_zc, 0)

    def fill_o(r, _):
      ov[pl.ds(r * L, L)] = jnp.ones((L,), jnp.float32)
      return 0
    lax.fori_loop(0, CH // L, fill_o, 0)

    for q in range(STRIPE // ZR):
      pltpu.sync_copy(zv, acc.at[pl.ds(s * STRIPE + q * ZR, ZR)])
      pltpu.sync_copy(zcv, accc.at[pl.ds(s * STRIPE + q * ZR, ZR)])
    plsc.subcore_barrier()

    def body(k, _):
      i = s + k * NS

      @pl.when(i < NCH)
      def _():
        pltpu.sync_copy(idx_hbm.at[pl.ds(i * 4, 4)], iv)
        pltpu.sync_copy(edges_hbm.at[i], pv)
        for j in range(4):
          for g in range(128 // L):
            v = iv[j, pl.ds(g * L, L)] - base
            ok = (v >= 0) & (v < NH)
            lv[j, pl.ds(g * L, L)] = jnp.where(ok, v, NH)
        for j in range(4):
          pltpu.sync_copy(pv.at[pl.ds(j * 128, 128)], acc.at[lv.at[j]],
                          add=True)
          pltpu.sync_copy(ov.at[pl.ds(j * 128, 128)], accc.at[lv.at[j]],
                          add=True)
      return 0

    lax.fori_loop(0, KMAX, body, 0)
    plsc.subcore_barrier()

    pltpu.sync_copy(acc.at[pl.ds(s * STRIPE, STRIPE)],
                    sums_out.at[c].at[pl.ds(s * STRIPE, STRIPE)])
    pltpu.sync_copy(accc.at[pl.ds(s * STRIPE, STRIPE)],
                    cnt_out.at[c].at[pl.ds(s * STRIPE, STRIPE)])

  return sk


def _node_mlp_body(sr, cr, ar, w1a, w1b, b1, w2, b2, w3, b3, outr):
  cnt = jnp.maximum(cr[0], 1.0)
  mean = sr[0] / cnt
  x = (jnp.dot(mean, w1a[...], preferred_element_type=jnp.float32)
       + jnp.dot(ar[...], w1b[...], preferred_element_type=jnp.float32)
       + b1[...])
  h = jnp.maximum(x, 0.0)
  h = jnp.maximum(
      jnp.dot(h, w2[...], preferred_element_type=jnp.float32) + b2[...], 0.0)
  outr[...] = jnp.dot(h, w3[...], preferred_element_type=jnp.float32) + b3[...]


@functools.lru_cache(maxsize=None)
def _make_node_mlp(N, NHP, D, Bn):
  G = N // Bn
  PB = G // NC
  full = lambda s: pl.BlockSpec(s, lambda i: (0, 0))
  return pl.pallas_call(
      _node_mlp_body,
      grid=(G,),
      in_specs=[
          pl.BlockSpec((1, Bn, D), lambda i: (i // PB, i % PB, 0)),
          pl.BlockSpec((1, Bn, 1), lambda i: (i // PB, i % PB, 0)),
          pl.BlockSpec((Bn, D), lambda i: (i, 0)),
          full((D, 128)), full((D, 128)), full((1, 128)),
          full((128, 64)), full((1, 64)),
          full((64, D)), full((1, D)),
      ],
      out_specs=pl.BlockSpec((Bn, D), lambda i: (i, 0)),
      out_shape=jax.ShapeDtypeStruct((N, D), jnp.float32),
  )


def kernel(bonds, bond_atom_1, bond_atom_2, atoms,
           We1, be1, We2, be2, We3, be3,
           Wv1, bv1, Wv2, bv2, Wv3, bv3):
  E, D = bonds.shape
  N = atoms.shape[0]

  idx1 = bond_atom_1.astype(jnp.int32).reshape(E // 128, 128)
  idx2 = bond_atom_2.astype(jnp.int32).reshape(E // 128, 128)

  EP = E // 4
  a1p, a2p = _make_gather(N, E, D)(atoms, idx1, idx2)
  a1p = a1p.reshape(EP, 4 * D)
  a2p = a2p.reshape(EP, 4 * D)
  bp = bonds.reshape(EP, 4 * D)

  w1 = jnp.concatenate(
      [_blockdiag4(We1[:D]), _blockdiag4(We1[D:2 * D]),
       _blockdiag4(We1[2 * D:])], axis=0)
  bonds_new_p = _make_edge_mlp(EP, 2000)(
      a1p, a2p, bp,
      w1, jnp.tile(be1, 4).reshape(1, -1),
      _blockdiag4(We2), jnp.tile(be2, 4).reshape(1, -1),
      _blockdiag4(We3), jnp.tile(be3, 4).reshape(1, -1))

  sums, cnt = _make_scatter(N, E, D)(
      bonds_new_p.reshape(E // 512, 512, D), idx2)
  NHP = sums.shape[1]
  bonds_new = bonds_new_p.reshape(E, D)

  atoms_new = _make_node_mlp(N, NHP, D, 1000)(
      sums, cnt.reshape(NC, NHP, 1), atoms,
      Wv1[:D], Wv1[D:], bv1.reshape(1, -1),
      Wv2, bv2.reshape(1, -1), Wv3, bv3.reshape(1, -1))

  return (atoms_new, bonds_new)
